# baseline (speedup 1.0000x reference)
import jax
import jax.numpy as jnp
from jax import lax
from jax.experimental import pallas as pl
from jax.experimental.pallas import tpu as pltpu
from jax.experimental.pallas import tpu_sc as plsc

E = 320000
D = 128
PW = D // 2
N = 10000
N2 = 2 * N
NC = 2
NS = 16
WPS = PW // (NC * NS)
CE = 2000
NCHUNK = E // CE
GPC = CE // 16
EPB = E // NS
PB = 160
NBUF = 2


def _sc_body(tab_hbm, idx_hbm, pt_hbm, p0_hbm, p1_hbm,
             tab_v, idx0, idx1, part0, part1, rbuf, sbuf,
             semp0, semp1, semi0, semi1):
    idxb = (idx0, idx1)
    partb = (part0, part1)
    semp = (semp0, semp1)
    semi = (semi0, semi1)

    cid = lax.axis_index("c")
    sid = lax.axis_index("s")

    slab = (cid * NS + sid) * (N2 * WPS // 64)
    pltpu.sync_copy(tab_hbm.at[pl.ds(slab, N2 * WPS // 64)], tab_v)

    pbase = (cid * NS + sid) * E

    def fire_idx(ci, b):
        pltpu.async_copy(idx_hbm.at[pl.ds(ci * CE, CE)], idxb[b], semi[b])

    def drain_idx(b):
        pltpu.make_async_copy(idx_hbm.at[pl.ds(0, CE)],
                              idxb[b], semi[b]).wait()

    def fire_part(ci, b):
        pltpu.async_copy(partb[b], pt_hbm.at[pl.ds(pbase + ci * CE, CE)],
                         semp[b])

    def drain_part(b):
        pltpu.make_async_copy(partb[b], pt_hbm.at[pl.ds(pbase, CE)],
                              semp[b]).wait()

    def compute(b):
        @pl.loop(0, GPC)
        def group_loop(g):
            w = idxb[b][pl.ds(g * 16, 16)]
            s_ids = (w & 0xFFFF) * WPS
            d_ids = lax.shift_right_logical(w, 16) * WPS
            acca = jnp.zeros((16,), jnp.float32)
            accb = jnp.zeros((16,), jnp.float32)
            for k in range(WPS):
                sf = s_ids + k
                df = d_ids + k
                si = plsc.load_gather(
                    tab_v, [lax.shift_right_logical(sf, 6), sf & 63])
                ti = plsc.load_gather(
                    tab_v, [lax.shift_right_logical(df, 6), df & 63])
                sbf = plsc.bitcast(si, jnp.bfloat16)
                tbf = plsc.bitcast(ti, jnp.bfloat16)
                q0, q1 = plsc.unpack(sbf * tbf,
                                     format=plsc.PackFormat.INTERLEAVED)
                acca = acca + q0
                accb = accb + q1
            partb[b][pl.ds(g * 16, 16)] = acca + accb

    fire_idx(0, 0)
    drain_idx(0)
    fire_idx(1, 1)

    @pl.loop(0, NCHUNK, step=NBUF)
    def chunk_loop(ci0):
        for b in range(NBUF):
            ci = ci0 + b
            bn = 1 - b

            @pl.when(ci >= NBUF)
            def _():
                drain_part(b)

            compute(b)
            fire_part(ci, b)

            @pl.when(ci + 2 < NCHUNK)
            def _():
                fire_idx(ci + 2, b)

            @pl.when(ci + 1 < NCHUNK)
            def _():
                drain_idx(bn)

    drain_part(0)
    drain_part(1)

    plsc.subcore_barrier()

    cbase = cid * NS * E
    span0 = sid * EPB

    @pl.loop(0, EPB // PB)
    def reduce_loop(p):
        off = span0 + p * PB
        for k in range(NS):
            pltpu.async_copy(
                pt_hbm.at[pl.ds(cbase + k * E + off, PB)],
                rbuf.at[pl.ds(k * PB, PB)], semp[0])
        for k in range(NS):
            pltpu.make_async_copy(
                pt_hbm.at[pl.ds(cbase + k * E + off, PB)],
                rbuf.at[pl.ds(k * PB, PB)], semp[0]).wait()

        @pl.loop(0, PB // 16)
        def sum_loop(j):
            acc = rbuf[pl.ds(j * 16, 16)]
            for k in range(1, NS):
                acc = acc + rbuf[pl.ds(k * PB + j * 16, 16)]
            sbuf[pl.ds(j * 16, 16)] = acc

        @pl.when(cid == 0)
        def _():
            pltpu.sync_copy(sbuf, p0_hbm.at[pl.ds(off, PB)])

        @pl.when(cid == 1)
        def _():
            pltpu.sync_copy(sbuf, p1_hbm.at[pl.ds(off, PB)])


def _sigmoid_body(p0_ref, p1_ref, out_ref):
    out_ref[...] = jax.nn.sigmoid(p0_ref[...] + p1_ref[...])


@jax.jit
def _edge_decoder(tab_sliced, idx_packed):
    mesh = plsc.VectorSubcoreMesh(
        core_axis_name="c", subcore_axis_name="s",
        num_cores=NC, num_subcores=NS)
    _, p0, p1 = pl.kernel(
        _sc_body,
        out_type=(jax.ShapeDtypeStruct((NC * NS * E,), jnp.float32),
                  jax.ShapeDtypeStruct((E,), jnp.float32),
                  jax.ShapeDtypeStruct((E,), jnp.float32)),
        mesh=mesh,
        compiler_params=pltpu.CompilerParams(
            needs_layout_passes=False, use_tc_tiling_on_sc=False),
        scratch_types=[
            pltpu.VMEM((N2 * WPS // 64, 64), jnp.int32),
            pltpu.VMEM((CE,), jnp.int32),
            pltpu.VMEM((CE,), jnp.int32),
            pltpu.VMEM((CE,), jnp.float32),
            pltpu.VMEM((CE,), jnp.float32),
            pltpu.VMEM((NS * PB,), jnp.float32),
            pltpu.VMEM((PB,), jnp.float32),
            pltpu.SemaphoreType.DMA,
            pltpu.SemaphoreType.DMA,
            pltpu.SemaphoreType.DMA,
            pltpu.SemaphoreType.DMA,
        ],
    )(tab_sliced, idx_packed)

    out = pl.pallas_call(
        _sigmoid_body,
        out_shape=jax.ShapeDtypeStruct((E // D, D), jnp.float32),
    )(p0.reshape(E // D, D), p1.reshape(E // D, D))
    return out.reshape(E)


def kernel(z_user, z_item, edge_index):
    zu_pk = lax.bitcast_convert_type(
        z_user.astype(jnp.bfloat16).reshape(N, PW, 2), jnp.int32)
    zi_pk = lax.bitcast_convert_type(
        z_item.astype(jnp.bfloat16).reshape(N, PW, 2), jnp.int32)
    zf_pk = jnp.stack([zu_pk, zi_pk], axis=1).reshape(N2, PW)
    tab_sliced = (zf_pk.reshape(N2, NC * NS, WPS)
                  .transpose(1, 0, 2).reshape(NC * NS * N2 * WPS // 64, 64))
    src2 = edge_index[0].astype(jnp.int32) * 2
    dst2 = edge_index[1].astype(jnp.int32) * 2 + 1
    idx_packed = src2 | (dst2 << 16)
    return _edge_decoder(tab_sliced, idx_packed)


# speedup vs baseline: 1.6391x; 1.6391x over previous
import jax
import jax.numpy as jnp
from jax import lax
from jax.experimental import pallas as pl
from jax.experimental.pallas import tpu as pltpu
from jax.experimental.pallas import tpu_sc as plsc

E = 320000
D = 128
PW = D // 2
N = 10000
NC = 2
NS = 16
NW = NC * NS
EPW = E // NW
C = 80
NCHUNK = EPW // C
G = C // 16
NBUF = 2


def _sc_body(zf_hbm, idx_hbm, out_hbm,
             idx0, idx1, rows0, rows1,
             out_v, semr0, semr1, semi0, semi1):
    idxb = (idx0, idx1)
    rows = (rows0, rows1)
    semr = (semr0, semr1)
    semi = (semi0, semi1)

    wid = lax.axis_index("s") * NC + lax.axis_index("c")
    base = wid * EPW

    def fire_idx(ci, b):
        cbase = (wid * NCHUNK + ci) * (2 * C)
        pltpu.async_copy(idx_hbm.at[pl.ds(cbase, 2 * C)], idxb[b], semi[b])

    def drain_idx(b):
        pltpu.make_async_copy(idx_hbm.at[pl.ds(0, 2 * C)],
                              idxb[b], semi[b]).wait()

    def fire_rows(b):
        pltpu.async_copy(zf_hbm.at[idxb[b]], rows[b], semr[b])

    def drain_rows(b):
        pltpu.make_async_copy(zf_hbm.at[idxb[b]], rows[b], semr[b]).wait()

    def compute(ci, b):
        for g in range(G):
            lanes = lax.iota(jnp.int32, 16) + (g * 16)
            lanes_d = lanes + C
            acca0 = jnp.zeros((16,), jnp.float32)
            accb0 = jnp.zeros((16,), jnp.float32)
            dv0 = jnp.zeros((16,), jnp.int32)

            @pl.loop(0, PW, init_carry=(acca0, accb0, dv0), unroll=8)
            def dot_loop(d, carry):
                acca, accb, dv = carry
                si = plsc.load_gather(rows[b], [lanes, dv])
                ti = plsc.load_gather(rows[b], [lanes_d, dv])
                sbf = plsc.bitcast(si, jnp.bfloat16)
                tbf = plsc.bitcast(ti, jnp.bfloat16)
                q0, q1 = plsc.unpack(sbf * tbf,
                                     format=plsc.PackFormat.INTERLEAVED)
                return acca + q0, accb + q1, dv + 1

            acca, accb, _ = dot_loop
            acc = acca + accb
            sig = 1.0 / (1.0 + jnp.exp(-acc))
            out_v[pl.ds(ci * C + g * 16, 16)] = sig

    fire_idx(0, 0)
    drain_idx(0)
    fire_rows(0)
    fire_idx(1, 1)

    @pl.loop(0, NCHUNK - 1, step=NBUF)
    def chunk_loop(ci0):
        for b in range(NBUF):
            ci = ci0 + b
            bn = 1 - b

            @pl.when(ci + 1 < NCHUNK)
            def _():
                drain_idx(bn)
                fire_rows(bn)

            drain_rows(b)

            @pl.when(ci + NBUF < NCHUNK)
            def _():
                fire_idx(ci + NBUF, b)

            compute(ci, b)

    drain_rows(0)
    compute(NCHUNK - 1, 0)

    pltpu.sync_copy(out_v, out_hbm.at[pl.ds(base, EPW)])


@jax.jit
def _edge_decoder(zf_pk, idx_blk):
    mesh = plsc.VectorSubcoreMesh(
        core_axis_name="c", subcore_axis_name="s",
        num_cores=NC, num_subcores=NS)
    return pl.kernel(
        _sc_body,
        out_type=jax.ShapeDtypeStruct((E,), jnp.float32),
        mesh=mesh,
        compiler_params=pltpu.CompilerParams(
            needs_layout_passes=False, use_tc_tiling_on_sc=False),
        scratch_types=[
            pltpu.VMEM((2 * C,), jnp.int32),
            pltpu.VMEM((2 * C,), jnp.int32),
            pltpu.VMEM((2 * C, PW), jnp.int32),
            pltpu.VMEM((2 * C, PW), jnp.int32),
            pltpu.VMEM((EPW,), jnp.float32),
            pltpu.SemaphoreType.DMA,
            pltpu.SemaphoreType.DMA,
            pltpu.SemaphoreType.DMA,
            pltpu.SemaphoreType.DMA,
        ],
    )(zf_pk, idx_blk)


def kernel(z_user, z_item, edge_index):
    zu_pk = lax.bitcast_convert_type(
        z_user.astype(jnp.bfloat16).reshape(N, PW, 2), jnp.int32)
    zi_pk = lax.bitcast_convert_type(
        z_item.astype(jnp.bfloat16).reshape(N, PW, 2), jnp.int32)
    zf_pk = jnp.stack([zu_pk, zi_pk], axis=1).reshape(2 * N, PW)
    src2 = edge_index[0].astype(jnp.int32) * 2
    dst2 = edge_index[1].astype(jnp.int32) * 2 + 1
    idx_blk = jnp.stack(
        [src2.reshape(NW, NCHUNK, C), dst2.reshape(NW, NCHUNK, C)],
        axis=2).reshape(NW * NCHUNK * 2 * C)
    return _edge_decoder(zf_pk, idx_blk)


# speedup vs baseline: 1.6990x; 1.0365x over previous
import jax
import jax.numpy as jnp
from jax import lax
from jax.experimental import pallas as pl
from jax.experimental.pallas import tpu as pltpu
from jax.experimental.pallas import tpu_sc as plsc

E = 320000
D = 128
PW = D // 2
N = 10000
NC = 2
NS = 16
NW = NC * NS
EPW = E // NW
C = 80
NCHUNK = EPW // C
G = C // 16
NBUF = 3


def _sc_body(zu_hbm, zi_hbm, src_hbm, dst_hbm, out_hbm,
             sidx0, sidx1, sidx2, didx0, didx1, didx2,
             srow0, srow1, srow2, drow0, drow1, drow2,
             out_v, semr0, semr1, semr2, semi0, semi1, semi2):
    sidx = (sidx0, sidx1, sidx2)
    didx = (didx0, didx1, didx2)
    srow = (srow0, srow1, srow2)
    drow = (drow0, drow1, drow2)
    semr = (semr0, semr1, semr2)
    semi = (semi0, semi1, semi2)

    wid = lax.axis_index("s") * NC + lax.axis_index("c")
    base = wid * EPW

    def fire_idx(ci, b):
        cbase = base + ci * C
        pltpu.async_copy(src_hbm.at[pl.ds(cbase, C)], sidx[b], semi[b])
        pltpu.async_copy(dst_hbm.at[pl.ds(cbase, C)], didx[b], semi[b])

    def drain_idx(b):
        pltpu.make_async_copy(src_hbm.at[pl.ds(0, C)], sidx[b], semi[b]).wait()
        pltpu.make_async_copy(dst_hbm.at[pl.ds(0, C)], didx[b], semi[b]).wait()

    def fire_rows(b):
        pltpu.async_copy(zu_hbm.at[sidx[b]], srow[b], semr[b])
        pltpu.async_copy(zi_hbm.at[didx[b]], drow[b], semr[b])

    def drain_rows(b):
        pltpu.make_async_copy(zu_hbm.at[sidx[b]], srow[b], semr[b]).wait()
        pltpu.make_async_copy(zi_hbm.at[didx[b]], drow[b], semr[b]).wait()

    def compute(ci, b):
        for g in range(G):
            lanes = lax.iota(jnp.int32, 16) + (g * 16)
            acca0 = jnp.zeros((16,), jnp.float32)
            accb0 = jnp.zeros((16,), jnp.float32)
            dv0 = jnp.zeros((16,), jnp.int32)

            @pl.loop(0, PW, init_carry=(acca0, accb0, dv0), unroll=8)
            def dot_loop(d, carry):
                acca, accb, dv = carry
                si = plsc.load_gather(srow[b], [lanes, dv])
                ti = plsc.load_gather(drow[b], [lanes, dv])
                sbf = plsc.bitcast(si, jnp.bfloat16)
                tbf = plsc.bitcast(ti, jnp.bfloat16)
                q0, q1 = plsc.unpack(sbf * tbf,
                                     format=plsc.PackFormat.INTERLEAVED)
                return acca + q0, accb + q1, dv + 1

            acca, accb, _ = dot_loop
            acc = acca + accb
            sig = 1.0 / (1.0 + jnp.exp(-acc))
            out_v[pl.ds(ci * C + g * 16, 16)] = sig

    fire_idx(0, 0)
    drain_idx(0)
    fire_rows(0)
    fire_idx(1, 1)
    drain_idx(1)
    fire_rows(1)
    fire_idx(2, 2)

    @pl.loop(0, NCHUNK - 2, step=NBUF)
    def chunk_loop(ci0):
        for b in range(NBUF):
            ci = ci0 + b
            b2 = (b + 2) % NBUF

            @pl.when(ci + 2 < NCHUNK)
            def _():
                drain_idx(b2)
                fire_rows(b2)

            drain_rows(b)

            @pl.when(ci + NBUF < NCHUNK)
            def _():
                fire_idx(ci + NBUF, b)

            compute(ci, b)

    drain_rows((NCHUNK - 2) % NBUF)
    compute(NCHUNK - 2, (NCHUNK - 2) % NBUF)
    drain_rows((NCHUNK - 1) % NBUF)
    compute(NCHUNK - 1, (NCHUNK - 1) % NBUF)

    pltpu.sync_copy(out_v, out_hbm.at[pl.ds(base, EPW)])


@jax.jit
def _edge_decoder(zu_pk, zi_pk, src_idx, dst_idx):
    mesh = plsc.VectorSubcoreMesh(
        core_axis_name="c", subcore_axis_name="s",
        num_cores=NC, num_subcores=NS)
    return pl.kernel(
        _sc_body,
        out_type=jax.ShapeDtypeStruct((E,), jnp.float32),
        mesh=mesh,
        compiler_params=pltpu.CompilerParams(
            needs_layout_passes=False, use_tc_tiling_on_sc=False),
        scratch_types=(
            [pltpu.VMEM((C,), jnp.int32)] * 6
            + [pltpu.VMEM((C, PW), jnp.int32)] * 6
            + [pltpu.VMEM((EPW,), jnp.float32)]
            + [pltpu.SemaphoreType.DMA] * 6
        ),
    )(zu_pk, zi_pk, src_idx, dst_idx)


def kernel(z_user, z_item, edge_index):
    zu_pk = lax.bitcast_convert_type(
        z_user.astype(jnp.bfloat16).reshape(N, PW, 2), jnp.int32)
    zi_pk = lax.bitcast_convert_type(
        z_item.astype(jnp.bfloat16).reshape(N, PW, 2), jnp.int32)
    src_idx = edge_index[0].astype(jnp.int32)
    dst_idx = edge_index[1].astype(jnp.int32)
    return _edge_decoder(zu_pk, zi_pk, src_idx, dst_idx)


# speedup vs baseline: 1.7006x; 1.0009x over previous
import jax
import jax.numpy as jnp
from jax import lax
from jax.experimental import pallas as pl
from jax.experimental.pallas import tpu as pltpu
from jax.experimental.pallas import tpu_sc as plsc

E = 320000
D = 128
PW = D // 2
N = 10000
NC = 2
NS = 16
NW = NC * NS
EPW = E // NW
C = 80
NCHUNK = EPW // C
G = C // 16
NBUF = 2


def _sc_body(zu_hbm, zi_hbm, src_hbm, dst_hbm, out_hbm,
             sidx0, sidx1, didx0, didx1, srow0, srow1, drow0, drow1,
             out_v, semr0, semr1, semi0, semi1):
    sidx = (sidx0, sidx1)
    didx = (didx0, didx1)
    srow = (srow0, srow1)
    drow = (drow0, drow1)
    semr = (semr0, semr1)
    semi = (semi0, semi1)

    wid = lax.axis_index("s") * NC + lax.axis_index("c")
    base = wid * EPW

    def fire_idx(ci, b):
        cbase = base + ci * C
        pltpu.async_copy(src_hbm.at[pl.ds(cbase, C)], sidx[b], semi[b])
        pltpu.async_copy(dst_hbm.at[pl.ds(cbase, C)], didx[b], semi[b])

    def drain_idx(b):
        pltpu.make_async_copy(src_hbm.at[pl.ds(0, C)], sidx[b], semi[b]).wait()
        pltpu.make_async_copy(dst_hbm.at[pl.ds(0, C)], didx[b], semi[b]).wait()

    def fire_rows(b):
        pltpu.async_copy(zu_hbm.at[sidx[b]], srow[b], semr[b])
        pltpu.async_copy(zi_hbm.at[didx[b]], drow[b], semr[b])

    def drain_rows(b):
        pltpu.make_async_copy(zu_hbm.at[sidx[b]], srow[b], semr[b]).wait()
        pltpu.make_async_copy(zi_hbm.at[didx[b]], drow[b], semr[b]).wait()

    def compute(ci, b):
        for g in range(G):
            lanes = lax.iota(jnp.int32, 16) + (g * 16)
            acca0 = jnp.zeros((16,), jnp.float32)
            accb0 = jnp.zeros((16,), jnp.float32)
            dv0 = jnp.zeros((16,), jnp.int32)

            @pl.loop(0, PW, init_carry=(acca0, accb0, dv0), unroll=8)
            def dot_loop(d, carry):
                acca, accb, dv = carry
                si = plsc.load_gather(srow[b], [lanes, dv])
                ti = plsc.load_gather(drow[b], [lanes, dv])
                sbf = plsc.bitcast(si, jnp.bfloat16)
                tbf = plsc.bitcast(ti, jnp.bfloat16)
                q0, q1 = plsc.unpack(sbf * tbf,
                                     format=plsc.PackFormat.INTERLEAVED)
                return acca + q0, accb + q1, dv + 1

            acca, accb, _ = dot_loop
            acc = acca + accb
            sig = 1.0 / (1.0 + jnp.exp(-acc))
            out_v[pl.ds(ci * C + g * 16, 16)] = sig

    fire_idx(0, 0)
    drain_idx(0)
    fire_rows(0)
    fire_idx(1, 1)

    @pl.loop(0, NCHUNK - 1, step=NBUF)
    def chunk_loop(ci0):
        for b in range(NBUF):
            ci = ci0 + b
            bn = 1 - b

            @pl.when(ci + 1 < NCHUNK)
            def _():
                drain_idx(bn)
                fire_rows(bn)

            drain_rows(b)

            @pl.when(ci + NBUF < NCHUNK)
            def _():
                fire_idx(ci + NBUF, b)

            compute(ci, b)

    drain_rows(0)
    compute(NCHUNK - 1, 0)

    pltpu.sync_copy(out_v, out_hbm.at[pl.ds(base, EPW)])


@jax.jit
def _edge_decoder(zu_pk, zi_pk, src_idx, dst_idx):
    mesh = plsc.VectorSubcoreMesh(
        core_axis_name="c", subcore_axis_name="s",
        num_cores=NC, num_subcores=NS)
    return pl.kernel(
        _sc_body,
        out_type=jax.ShapeDtypeStruct((E,), jnp.float32),
        mesh=mesh,
        compiler_params=pltpu.CompilerParams(
            needs_layout_passes=False, use_tc_tiling_on_sc=False),
        scratch_types=[
            pltpu.VMEM((C,), jnp.int32),
            pltpu.VMEM((C,), jnp.int32),
            pltpu.VMEM((C,), jnp.int32),
            pltpu.VMEM((C,), jnp.int32),
            pltpu.VMEM((C, PW), jnp.int32),
            pltpu.VMEM((C, PW), jnp.int32),
            pltpu.VMEM((C, PW), jnp.int32),
            pltpu.VMEM((C, PW), jnp.int32),
            pltpu.VMEM((EPW,), jnp.float32),
            pltpu.SemaphoreType.DMA,
            pltpu.SemaphoreType.DMA,
            pltpu.SemaphoreType.DMA,
            pltpu.SemaphoreType.DMA,
        ],
    )(zu_pk, zi_pk, src_idx, dst_idx)


def kernel(z_user, z_item, edge_index):
    zu_pk = lax.bitcast_convert_type(
        z_user.astype(jnp.bfloat16).reshape(N, PW, 2), jnp.int32)
    zi_pk = lax.bitcast_convert_type(
        z_item.astype(jnp.bfloat16).reshape(N, PW, 2), jnp.int32)
    src_idx = edge_index[0].astype(jnp.int32)
    dst_idx = edge_index[1].astype(jnp.int32)
    return _edge_decoder(zu_pk, zi_pk, src_idx, dst_idx)
